# Initial kernel scaffold; baseline (speedup 1.0000x reference)
import jax
import jax.numpy as jnp
from jax.experimental import pallas as pl


def kernel(pred_logits, pred_boxes, orig_target_sizes):
    raise NotImplementedError("write your pallas kernel here")


# speedup vs baseline: 17.9725x; 17.9725x over previous
import jax
import jax.numpy as jnp
from jax.experimental import pallas as pl

_NUM_CLASSES = 80
_TOPK = 300
_QBLK = 2000


def _qmax_body(logits_ref, out_ref):
    x = logits_ref[...]
    out_ref[...] = jnp.max(x, axis=-1).reshape(1, 1, 8, _QBLK // 8)


def _sig_body(logits_ref, out_ref):
    out_ref[...] = jax.nn.sigmoid(logits_ref[...])


def _box_body(boxes_ref, size_ref, boxes_out_ref):
    b = boxes_ref[0]
    s = size_ref[0]
    xy = b[:, 0:2]
    wh = b[:, 2:4]
    mins = xy - 0.5 * wh
    maxs = xy + 0.5 * wh
    scale = jnp.concatenate([s, s], axis=-1)
    boxes_out_ref[...] = (jnp.concatenate([mins, maxs], axis=-1) * scale)[None]


def kernel(pred_logits, pred_boxes, orig_target_sizes):
    B, NQ, C = pred_logits.shape

    nblk = NQ // _QBLK
    qmax = pl.pallas_call(
        _qmax_body,
        grid=(B, nblk),
        in_specs=[pl.BlockSpec((1, _QBLK, C), lambda b, q: (b, q, 0))],
        out_specs=pl.BlockSpec((1, 1, 8, _QBLK // 8), lambda b, q: (b, q, 0, 0)),
        out_shape=jax.ShapeDtypeStruct((B, nblk, 8, _QBLK // 8), jnp.float32),
    )(pred_logits)
    qmax = qmax.reshape(B, NQ)

    _, cand_q = jax.lax.top_k(qmax, _TOPK)
    cand_q = jnp.sort(cand_q, axis=1)

    cand_logits = jnp.take_along_axis(
        pred_logits, cand_q[..., None], axis=1
    )

    cand_scores = pl.pallas_call(
        _sig_body,
        grid=(B,),
        in_specs=[pl.BlockSpec((1, _TOPK, C), lambda b: (b, 0, 0))],
        out_specs=pl.BlockSpec((1, _TOPK, C), lambda b: (b, 0, 0)),
        out_shape=jax.ShapeDtypeStruct((B, _TOPK, C), jnp.float32),
    )(cand_logits)

    scores, top_idx = jax.lax.top_k(cand_scores.reshape(B, _TOPK * C), _TOPK)
    labels = top_idx % C
    query_index = jnp.take_along_axis(cand_q, top_idx // C, axis=1)

    boxes_g = jnp.take_along_axis(
        pred_boxes, query_index[..., None], axis=1
    )

    boxes = pl.pallas_call(
        _box_body,
        grid=(B,),
        in_specs=[
            pl.BlockSpec((1, _TOPK, 4), lambda b: (b, 0, 0)),
            pl.BlockSpec((1, 1, 2), lambda b: (b, 0, 0)),
        ],
        out_specs=pl.BlockSpec((1, _TOPK, 4), lambda b: (b, 0, 0)),
        out_shape=jax.ShapeDtypeStruct((B, _TOPK, 4), jnp.float32),
    )(boxes_g, orig_target_sizes[:, None, :])

    return (labels, boxes, scores)


# speedup vs baseline: 24.4988x; 1.3631x over previous
import jax
import jax.numpy as jnp
from jax.experimental import pallas as pl

_NUM_CLASSES = 80
_TOPK = 300
_QBLK = 2000


def _qmax_body(logits_ref, out_ref):
    x = logits_ref[...]
    out_ref[...] = jnp.max(x, axis=-1).reshape(1, 1, 8, _QBLK // 8)


def _sig_body(logits_ref, out_ref):
    out_ref[...] = jax.nn.sigmoid(logits_ref[...])


def _box_body(boxes_ref, size_ref, boxes_out_ref):
    b = boxes_ref[0]
    s = size_ref[0]
    xy = b[:, 0:2]
    wh = b[:, 2:4]
    mins = xy - 0.5 * wh
    maxs = xy + 0.5 * wh
    scale = jnp.concatenate([s, s], axis=-1)
    boxes_out_ref[...] = (jnp.concatenate([mins, maxs], axis=-1) * scale)[None]


def kernel(pred_logits, pred_boxes, orig_target_sizes):
    B, NQ, C = pred_logits.shape

    nblk = NQ // _QBLK
    qmax = pl.pallas_call(
        _qmax_body,
        grid=(B, nblk),
        in_specs=[pl.BlockSpec((1, _QBLK, C), lambda b, q: (b, q, 0))],
        out_specs=pl.BlockSpec((1, 1, 8, _QBLK // 8), lambda b, q: (b, q, 0, 0)),
        out_shape=jax.ShapeDtypeStruct((B, nblk, 8, _QBLK // 8), jnp.float32),
    )(pred_logits)
    qmax = qmax.reshape(B, NQ)

    ncand = _TOPK + 20
    _, cand_q = jax.lax.approx_max_k(qmax, ncand, recall_target=1.0)
    cand_q = jnp.sort(cand_q, axis=1)

    cand_logits = jnp.take_along_axis(
        pred_logits, cand_q[..., None], axis=1
    )

    cand_scores = pl.pallas_call(
        _sig_body,
        grid=(B,),
        in_specs=[pl.BlockSpec((1, ncand, C), lambda b: (b, 0, 0))],
        out_specs=pl.BlockSpec((1, ncand, C), lambda b: (b, 0, 0)),
        out_shape=jax.ShapeDtypeStruct((B, ncand, C), jnp.float32),
    )(cand_logits)

    nsup = 512
    sup_vals, sup_idx = jax.lax.approx_max_k(
        cand_scores.reshape(B, ncand * C), nsup, recall_target=1.0
    )
    order = jnp.lexsort((sup_idx, -sup_vals), axis=-1)[:, :_TOPK]
    scores = jnp.take_along_axis(sup_vals, order, axis=1)
    top_idx = jnp.take_along_axis(sup_idx, order, axis=1)
    labels = top_idx % C
    query_index = jnp.take_along_axis(cand_q, top_idx // C, axis=1)

    boxes_g = jnp.take_along_axis(
        pred_boxes, query_index[..., None], axis=1
    )

    boxes = pl.pallas_call(
        _box_body,
        grid=(B,),
        in_specs=[
            pl.BlockSpec((1, _TOPK, 4), lambda b: (b, 0, 0)),
            pl.BlockSpec((1, 1, 2), lambda b: (b, 0, 0)),
        ],
        out_specs=pl.BlockSpec((1, _TOPK, 4), lambda b: (b, 0, 0)),
        out_shape=jax.ShapeDtypeStruct((B, _TOPK, 4), jnp.float32),
    )(boxes_g, orig_target_sizes[:, None, :])

    return (labels, boxes, scores)
